# baseline (speedup 1.0000x reference)
import jax
import jax.numpy as jnp
from jax import lax
from jax.experimental import pallas as pl
from jax.experimental.pallas import tpu as pltpu
from jax.experimental.pallas import tpu_sc as plsc

_S_SPARSE = 128
_S_FULL = 4096
_D_DET = 1024
_NW = 16
_RPW = _S_SPARSE // _NW
_CH = 256
_NCH = _S_FULL // _CH


def _tc_copy_body(pred_vmem, out_vmem):
    out_vmem[...] = pred_vmem[...]


def _sc_scatter_body(sparse_hbm, out_hbm, srows, sem_l, sem_s):
    wid = lax.axis_index("s")
    pltpu.async_copy(sparse_hbm.at[pl.ds(wid * _RPW, _RPW)], srows, sem_l).wait()
    handles = []
    for j in range(_RPW):
        b = wid * _RPW + j
        vi = (b * 4095) // 127
        handles.append(
            pltpu.async_copy(srows.at[pl.ds(j, 1)], out_hbm.at[pl.ds(vi, 1)], sem_s))
    for h in handles:
        h.wait()


def kernel(sinogram_sparse, sinogram_pred):
    sp = sinogram_sparse.reshape(_S_SPARSE, _D_DET)
    pr = sinogram_pred.reshape(_S_FULL, _D_DET)
    copied = pl.pallas_call(
        _tc_copy_body,
        grid=(_NCH,),
        in_specs=[pl.BlockSpec((_CH, _D_DET), lambda i: (i, 0))],
        out_specs=pl.BlockSpec((_CH, _D_DET), lambda i: (i, 0)),
        out_shape=jax.ShapeDtypeStruct((_S_FULL, _D_DET), jnp.float32),
    )(pr)
    out_ref = jax.new_ref(copied)
    pl.kernel(
        _sc_scatter_body,
        out_type=(),
        mesh=plsc.VectorSubcoreMesh(
            core_axis_name="c", subcore_axis_name="s", num_cores=1),
        scratch_types=[
            pltpu.VMEM((_RPW, _D_DET), jnp.float32),
            pltpu.SemaphoreType.DMA,
            pltpu.SemaphoreType.DMA,
        ],
    )(sp, out_ref)
    return out_ref[...][None, None, :, :]


# speedup vs baseline: 1.1441x; 1.1441x over previous
import jax
import jax.numpy as jnp
from jax import lax
from jax.experimental import pallas as pl
from jax.experimental.pallas import tpu as pltpu
from jax.experimental.pallas import tpu_sc as plsc

_S_SPARSE = 128
_S_FULL = 4096
_D_DET = 1024
_NW = 16
_RPW = _S_SPARSE // _NW
_CH = 1024
_NCH = _S_FULL // _CH


def _tc_copy_body(pred_vmem, out_vmem):
    out_vmem[...] = pred_vmem[...]


def _sc_scatter_body(sparse_hbm, out_hbm, srows, sem_l, sem_s):
    wid = lax.axis_index("s")
    pltpu.async_copy(sparse_hbm.at[pl.ds(wid * _RPW, _RPW)], srows, sem_l).wait()
    handles = []
    for j in range(_RPW):
        b = wid * _RPW + j
        vi = (b * 4095) // 127
        handles.append(
            pltpu.async_copy(srows.at[pl.ds(j, 1)], out_hbm.at[pl.ds(vi, 1)], sem_s))
    for h in handles:
        h.wait()


def kernel(sinogram_sparse, sinogram_pred):
    sp = sinogram_sparse.reshape(_S_SPARSE, _D_DET)
    pr = sinogram_pred.reshape(_S_FULL, _D_DET)
    copied = pl.pallas_call(
        _tc_copy_body,
        grid=(_NCH,),
        in_specs=[pl.BlockSpec((_CH, _D_DET), lambda i: (i, 0))],
        out_specs=pl.BlockSpec((_CH, _D_DET), lambda i: (i, 0)),
        out_shape=jax.ShapeDtypeStruct((_S_FULL, _D_DET), jnp.float32),
    )(pr)
    out_ref = jax.new_ref(copied)
    pl.kernel(
        _sc_scatter_body,
        out_type=(),
        mesh=plsc.VectorSubcoreMesh(
            core_axis_name="c", subcore_axis_name="s", num_cores=1),
        scratch_types=[
            pltpu.VMEM((_RPW, _D_DET), jnp.float32),
            pltpu.SemaphoreType.DMA,
            pltpu.SemaphoreType.DMA,
        ],
    )(sp, out_ref)
    return out_ref[...][None, None, :, :]


# speedup vs baseline: 1.2209x; 1.0672x over previous
import jax
import jax.numpy as jnp
from jax import lax
from jax.experimental import pallas as pl
from jax.experimental.pallas import tpu as pltpu
from jax.experimental.pallas import tpu_sc as plsc

_S_SPARSE = 128
_S_FULL = 4096
_D_DET = 1024
_NW = 16
_RPW = _S_SPARSE // _NW
_CH = 2048
_NCH = _S_FULL // _CH


def _tc_copy_body(pred_vmem, out_vmem):
    out_vmem[...] = pred_vmem[...]


def _sc_scatter_body(sparse_hbm, out_hbm, srows, sem_l, sem_s):
    wid = lax.axis_index("s")
    pltpu.async_copy(sparse_hbm.at[pl.ds(wid * _RPW, _RPW)], srows, sem_l).wait()
    handles = []
    for j in range(_RPW):
        b = wid * _RPW + j
        vi = (b * 4095) // 127
        handles.append(
            pltpu.async_copy(srows.at[pl.ds(j, 1)], out_hbm.at[pl.ds(vi, 1)], sem_s))
    for h in handles:
        h.wait()


def kernel(sinogram_sparse, sinogram_pred):
    sp = sinogram_sparse.reshape(_S_SPARSE, _D_DET)
    pr = sinogram_pred.reshape(_S_FULL, _D_DET)
    copied = pl.pallas_call(
        _tc_copy_body,
        grid=(_NCH,),
        in_specs=[pl.BlockSpec((_CH, _D_DET), lambda i: (i, 0))],
        out_specs=pl.BlockSpec((_CH, _D_DET), lambda i: (i, 0)),
        out_shape=jax.ShapeDtypeStruct((_S_FULL, _D_DET), jnp.float32),
    )(pr)
    out_ref = jax.new_ref(copied)
    pl.kernel(
        _sc_scatter_body,
        out_type=(),
        mesh=plsc.VectorSubcoreMesh(
            core_axis_name="c", subcore_axis_name="s", num_cores=1),
        scratch_types=[
            pltpu.VMEM((_RPW, _D_DET), jnp.float32),
            pltpu.SemaphoreType.DMA,
            pltpu.SemaphoreType.DMA,
        ],
    )(sp, out_ref)
    return out_ref[...][None, None, :, :]
